# Initial kernel scaffold; baseline (speedup 1.0000x reference)
import functools

import jax
import jax.numpy as jnp
from jax import lax
from jax.experimental import pallas as pl
from jax.experimental.pallas import tpu as pltpu
from jax.experimental.pallas import tpu_sc as plsc

EMBED_DIM = 32
LANE = 128
CHUNK = 1024
NBUF = 2


def _sc_info():
    try:
        info = plsc.get_sparse_core_info()
        return info.num_cores, info.num_subcores
    except Exception:
        return 2, 16


@functools.cache
def _make_gather(B, V):
    NC, NS = _sc_info()
    NW = NC * NS
    b_per_w = B // NW
    n_chunks = b_per_w // CHUNK
    n_pairs = n_chunks // NBUF
    JPC = CHUNK // LANE
    assert B % (NW * CHUNK) == 0 and n_chunks % NBUF == 0

    mesh = plsc.VectorSubcoreMesh(core_axis_name="c", subcore_axis_name="s")

    @functools.partial(
        pl.kernel,
        mesh=mesh,
        out_type=jax.ShapeDtypeStruct((B, EMBED_DIM), jnp.float32),
        scratch_types=[
            pltpu.VMEM((NBUF, JPC, LANE), jnp.int32),
            pltpu.VMEM((NBUF, CHUNK, EMBED_DIM), jnp.float32),
            pltpu.SemaphoreType.DMA,
            pltpu.SemaphoreType.DMA,
            pltpu.SemaphoreType.DMA,
        ],
    )
    def gather_kernel(idx_hbm, table_hbm, out_hbm, idx_v, rows_v, gsem, wsem0, wsem1):
        wid = lax.axis_index("s") * NC + lax.axis_index("c")
        wbase = wid * b_per_w
        wrow = wid * (b_per_w // LANE)
        wsems = (wsem0, wsem1)

        def pair_body(g, carry):
            for b in range(NBUF):
                c = g * NBUF + b
                base = wbase + c * CHUNK

                @pl.when(g > 0)
                def _wait_prev_write():
                    pltpu.make_async_copy(
                        rows_v.at[b], out_hbm.at[pl.ds(base, CHUNK)], wsems[b]
                    ).wait()

                pltpu.sync_copy(
                    idx_hbm.at[pl.ds(wrow + c * JPC, JPC)], idx_v.at[b]
                )
                copies = [
                    pltpu.async_copy(
                        table_hbm.at[idx_v.at[b, j]],
                        rows_v.at[b, pl.ds(j * LANE, LANE)],
                        gsem,
                    )
                    for j in range(JPC)
                ]
                for cp in copies:
                    cp.wait()
                pltpu.async_copy(
                    rows_v.at[b], out_hbm.at[pl.ds(base, CHUNK)], wsems[b]
                )
            return carry

        lax.fori_loop(0, n_pairs, pair_body, 0)

        for b in range(NBUF):
            base = wbase + (n_chunks - NBUF + b) * CHUNK
            pltpu.make_async_copy(
                rows_v.at[b], out_hbm.at[pl.ds(base, CHUNK)], wsems[b]
            ).wait()

    return gather_kernel


def kernel(time, table):
    B = time.shape[0] * time.shape[1]
    idx = time.reshape(B // LANE, LANE).astype(jnp.int32)
    return _make_gather(B, table.shape[0])(idx, table)


# speedup vs baseline: 7.0184x; 7.0184x over previous
import functools

import jax
import jax.numpy as jnp
from jax import lax
from jax.experimental import pallas as pl
from jax.experimental.pallas import tpu as pltpu
from jax.experimental.pallas import tpu_sc as plsc

EMBED_DIM = 32
LANE = 128
CHUNK = 1024
NBUF = 2


def _sc_info():
    try:
        info = plsc.get_sparse_core_info()
        return info.num_cores, info.num_subcores
    except Exception:
        return 2, 16


@functools.cache
def _make_gather(B, V):
    NC, NS = _sc_info()
    NW = NC * NS
    b_per_w = B // NW
    n_chunks = b_per_w // CHUNK
    n_pairs = n_chunks // NBUF
    JPC = CHUNK // LANE
    assert B % (NW * CHUNK) == 0 and n_chunks % NBUF == 0

    mesh = plsc.VectorSubcoreMesh(core_axis_name="c", subcore_axis_name="s")

    @functools.partial(
        pl.kernel,
        mesh=mesh,
        out_type=jax.ShapeDtypeStruct((B, EMBED_DIM), jnp.float32),
        scratch_types=[
            pltpu.VMEM((NBUF, JPC, LANE), jnp.int32),
            pltpu.VMEM((NBUF, CHUNK, EMBED_DIM), jnp.float32),
            pltpu.SemaphoreType.DMA,
            pltpu.SemaphoreType.DMA,
            pltpu.SemaphoreType.DMA,
        ],
        compiler_params=pltpu.CompilerParams(use_tc_tiling_on_sc=False),
    )
    def gather_kernel(idx_hbm, table_hbm, out_hbm, idx_v, rows_v, gsem, wsem0, wsem1):
        wid = lax.axis_index("s") * NC + lax.axis_index("c")
        wbase = wid * b_per_w
        wrow = wid * (b_per_w // LANE)
        wsems = (wsem0, wsem1)

        def pair_body(g, carry):
            for b in range(NBUF):
                c = g * NBUF + b
                base = wbase + c * CHUNK

                @pl.when(g > 0)
                def _wait_prev_write():
                    pltpu.make_async_copy(
                        rows_v.at[b], out_hbm.at[pl.ds(base, CHUNK)], wsems[b]
                    ).wait()

                pltpu.sync_copy(
                    idx_hbm.at[pl.ds(wrow + c * JPC, JPC)], idx_v.at[b]
                )
                copies = [
                    pltpu.async_copy(
                        table_hbm.at[idx_v.at[b, j]],
                        rows_v.at[b, pl.ds(j * LANE, LANE)],
                        gsem,
                    )
                    for j in range(JPC)
                ]
                for cp in copies:
                    cp.wait()
                pltpu.async_copy(
                    rows_v.at[b], out_hbm.at[pl.ds(base, CHUNK)], wsems[b]
                )
            return carry

        lax.fori_loop(0, n_pairs, pair_body, 0)

        for b in range(NBUF):
            base = wbase + (n_chunks - NBUF + b) * CHUNK
            pltpu.make_async_copy(
                rows_v.at[b], out_hbm.at[pl.ds(base, CHUNK)], wsems[b]
            ).wait()

    return gather_kernel


def kernel(time, table):
    B = time.shape[0] * time.shape[1]
    idx = time.reshape(B // LANE, LANE).astype(jnp.int32)
    return _make_gather(B, table.shape[0])(idx, table)


# speedup vs baseline: 7.0204x; 1.0003x over previous
import functools

import jax
import jax.numpy as jnp
from jax import lax
from jax.experimental import pallas as pl
from jax.experimental.pallas import tpu as pltpu
from jax.experimental.pallas import tpu_sc as plsc

EMBED_DIM = 32
LANE = 128
CHUNK = 1024
NBUF = 2


def _sc_info():
    try:
        info = plsc.get_sparse_core_info()
        return info.num_cores, info.num_subcores
    except Exception:
        return 2, 16


@functools.cache
def _make_gather(B, V):
    NC, NS = _sc_info()
    NW = NC * NS
    b_per_w = B // NW
    n_chunks = b_per_w // CHUNK
    n_pairs = n_chunks // NBUF
    JPC = CHUNK // LANE
    assert B % (NW * CHUNK) == 0 and n_chunks % NBUF == 0

    mesh = plsc.VectorSubcoreMesh(core_axis_name="c", subcore_axis_name="s")

    @functools.partial(
        pl.kernel,
        mesh=mesh,
        out_type=jax.ShapeDtypeStruct((B, EMBED_DIM), jnp.float32),
        scratch_types=[
            pltpu.VMEM((NBUF, JPC, LANE), jnp.int32),
            pltpu.VMEM((NBUF, CHUNK, EMBED_DIM), jnp.float32),
            pltpu.SemaphoreType.DMA,
            pltpu.SemaphoreType.DMA,
            pltpu.SemaphoreType.DMA,
            pltpu.SemaphoreType.DMA,
        ],
        compiler_params=pltpu.CompilerParams(use_tc_tiling_on_sc=False),
    )
    def gather_kernel(
        idx_hbm, table_hbm, out_hbm, idx_v, rows_v, gsem0, gsem1, wsem0, wsem1
    ):
        wid = lax.axis_index("s") * NC + lax.axis_index("c")
        wbase = wid * b_per_w
        wrow = wid * (b_per_w // LANE)
        gsems = (gsem0, gsem1)
        wsems = (wsem0, wsem1)

        def fire(c, b):
            pltpu.sync_copy(idx_hbm.at[pl.ds(wrow + c * JPC, JPC)], idx_v.at[b])
            for j in range(JPC):
                pltpu.async_copy(
                    table_hbm.at[idx_v.at[b, j]],
                    rows_v.at[b, pl.ds(j * LANE, LANE)],
                    gsems[b],
                )

        def drain_write(c, b):
            for j in range(JPC):
                pltpu.make_async_copy(
                    table_hbm.at[pl.ds(0, LANE)],
                    rows_v.at[b, pl.ds(j * LANE, LANE)],
                    gsems[b],
                ).wait()
            pltpu.async_copy(
                rows_v.at[b], out_hbm.at[pl.ds(wbase + c * CHUNK, CHUNK)], wsems[b]
            )

        def wait_write(c, b):
            pltpu.make_async_copy(
                rows_v.at[b], out_hbm.at[pl.ds(wbase + c * CHUNK, CHUNK)], wsems[b]
            ).wait()

        def pair_body(g, carry):
            @pl.when(g > 0)
            def _finish_prev():
                drain_write(g * 2 - 1, 1)
                wait_write(g * 2 - 2, 0)

            fire(g * 2, 0)

            drain_write(g * 2, 0)

            @pl.when(g > 0)
            def _reclaim():
                wait_write(g * 2 - 1, 1)

            fire(g * 2 + 1, 1)
            return carry

        lax.fori_loop(0, n_pairs, pair_body, 0)

        last = n_chunks - 1
        drain_write(last, 1)
        wait_write(last - 1, 0)
        wait_write(last, 1)

    return gather_kernel


def kernel(time, table):
    B = time.shape[0] * time.shape[1]
    idx = time.reshape(B // LANE, LANE).astype(jnp.int32)
    return _make_gather(B, table.shape[0])(idx, table)


# speedup vs baseline: 7.0359x; 1.0022x over previous
import functools

import jax
import jax.numpy as jnp
from jax import lax
from jax.experimental import pallas as pl
from jax.experimental.pallas import tpu as pltpu
from jax.experimental.pallas import tpu_sc as plsc

EMBED_DIM = 32
LANE = 128
CHUNK = 1024
NBUF = 2


def _sc_info():
    try:
        info = plsc.get_sparse_core_info()
        return info.num_cores, info.num_subcores
    except Exception:
        return 2, 16


@functools.cache
def _make_gather(B, V):
    NC, NS = _sc_info()
    NW = NC * NS
    b_per_w = B // NW
    n_chunks = b_per_w // CHUNK
    n_pairs = n_chunks // NBUF
    JPC = CHUNK // LANE
    assert B % (NW * CHUNK) == 0 and n_chunks % NBUF == 0

    mesh = plsc.VectorSubcoreMesh(core_axis_name="c", subcore_axis_name="s")

    @functools.partial(
        pl.kernel,
        mesh=mesh,
        out_type=jax.ShapeDtypeStruct((B, EMBED_DIM), jnp.float32),
        scratch_types=[
            pltpu.VMEM((NBUF, CHUNK), jnp.int32),
            pltpu.VMEM((NBUF, CHUNK, EMBED_DIM), jnp.float32),
            pltpu.SemaphoreType.DMA,
            pltpu.SemaphoreType.DMA,
            pltpu.SemaphoreType.DMA,
            pltpu.SemaphoreType.DMA,
        ],
        compiler_params=pltpu.CompilerParams(use_tc_tiling_on_sc=False),
    )
    def gather_kernel(
        idx_hbm, table_hbm, out_hbm, idx_v, rows_v, gsem0, gsem1, wsem0, wsem1
    ):
        wid = lax.axis_index("s") * NC + lax.axis_index("c")
        wbase = wid * b_per_w
        wrow = wid * (b_per_w // LANE)
        gsems = (gsem0, gsem1)
        wsems = (wsem0, wsem1)

        def fire(c, b):
            pltpu.sync_copy(idx_hbm.at[pl.ds(wbase + c * CHUNK, CHUNK)], idx_v.at[b])
            pltpu.async_copy(
                table_hbm.at[idx_v.at[b]],
                rows_v.at[b],
                gsems[b],
            )

        def drain_write(c, b):
            pltpu.make_async_copy(
                table_hbm.at[pl.ds(0, CHUNK)],
                rows_v.at[b],
                gsems[b],
            ).wait()
            pltpu.async_copy(
                rows_v.at[b], out_hbm.at[pl.ds(wbase + c * CHUNK, CHUNK)], wsems[b]
            )

        def wait_write(c, b):
            pltpu.make_async_copy(
                rows_v.at[b], out_hbm.at[pl.ds(wbase + c * CHUNK, CHUNK)], wsems[b]
            ).wait()

        def pair_body(g, carry):
            @pl.when(g > 0)
            def _finish_prev():
                drain_write(g * 2 - 1, 1)
                wait_write(g * 2 - 2, 0)

            fire(g * 2, 0)

            drain_write(g * 2, 0)

            @pl.when(g > 0)
            def _reclaim():
                wait_write(g * 2 - 1, 1)

            fire(g * 2 + 1, 1)
            return carry

        lax.fori_loop(0, n_pairs, pair_body, 0)

        last = n_chunks - 1
        drain_write(last, 1)
        wait_write(last - 1, 0)
        wait_write(last, 1)

    return gather_kernel


def kernel(time, table):
    B = time.shape[0] * time.shape[1]
    idx = time.reshape(B).astype(jnp.int32)
    return _make_gather(B, table.shape[0])(idx, table)
